# baseline (speedup 1.0000x reference)
import functools

import jax
import jax.numpy as jnp
from jax import lax
from jax.experimental import pallas as pl
from jax.experimental.pallas import tpu as pltpu
from jax.experimental.pallas import tpu_sc as plsc

N = 10000
E = 160000

NT = 16
NC = 2
K = 128
W = 128
EPAD = 163840
CH = EPAD // (NT * K)
CH2 = EPAD // (NT * NC * K)
NA = 10112
RPT = NA // NT
WPT = 624
TAIL = N - NT * WPT

_MESH = dict(core_axis_name="c", subcore_axis_name="s")


def _edge_pipeline(table, acc, src_v, dst_v, gbuf0, gbuf1,
                   sg0, sg1, ss0, ss1, nch):
    def fire_g(j, gbuf, sem):
        pltpu.async_copy(table.at[src_v.at[j]], gbuf, sem)

    def wait_g(j, gbuf, sem):
        pltpu.make_async_copy(table.at[src_v.at[j]], gbuf, sem).wait()

    def fire_s(j, gbuf, sem):
        pltpu.async_copy(gbuf, acc.at[dst_v.at[j]], sem, add=True)

    def wait_s(j, gbuf, sem):
        pltpu.make_async_copy(gbuf, acc.at[dst_v.at[j]], sem).wait()

    fire_g(0, gbuf0, sg0)

    def body(i, carry):
        j = 2 * i
        wait_g(j, gbuf0, sg0)
        fire_s(j, gbuf0, ss0)

        @pl.when(i > 0)
        def _():
            wait_s(j - 1, gbuf1, ss1)

        fire_g(j + 1, gbuf1, sg1)
        wait_g(j + 1, gbuf1, sg1)
        fire_s(j + 1, gbuf1, ss1)
        wait_s(j, gbuf0, ss0)

        @pl.when(i < nch // 2 - 1)
        def _():
            fire_g(j + 2, gbuf0, sg0)

        return carry

    lax.fori_loop(0, nch // 2, body, 0)
    wait_s(nch - 1, gbuf1, ss1)


def _zero_acc(zer_hbm, acc, s):
    pltpu.sync_copy(zer_hbm, acc.at[pl.ds(s * RPT, RPT)])


def _writeout(acc, o, s):
    w0 = s * WPT
    pltpu.sync_copy(acc.at[pl.ds(w0, WPT)], o.at[pl.ds(w0, WPT)])

    @pl.when(s == NT - 1)
    def _():
        pltpu.sync_copy(acc.at[pl.ds(NT * WPT, TAIL)],
                        o.at[pl.ds(NT * WPT, TAIL)])


def _make_segsum_colsplit():
    mesh = plsc.VectorSubcoreMesh(**_MESH)

    @functools.partial(
        pl.kernel,
        mesh=mesh,
        out_type=(
            jax.ShapeDtypeStruct((N, W), jnp.float32),
            jax.ShapeDtypeStruct((N, W), jnp.float32),
        ),
        scratch_types=[
            pltpu.VMEM_SHARED((NA, W), jnp.float32),
            pltpu.VMEM((CH2, K), jnp.int32),
            pltpu.VMEM((CH2, K), jnp.int32),
            pltpu.VMEM((K, W), jnp.float32),
            pltpu.VMEM((K, W), jnp.float32),
            pltpu.SemaphoreType.DMA,
            pltpu.SemaphoreType.DMA,
            pltpu.SemaphoreType.DMA,
            pltpu.SemaphoreType.DMA,
        ],
    )
    def seg(y_a, y_b, src_hbm, dst_hbm, zer_hbm,
            o_a, o_b, acc, src_v, dst_v, gbuf0, gbuf1, sg0, sg1, ss0, ss1):
        c = lax.axis_index("c")
        s = lax.axis_index("s")

        _zero_acc(zer_hbm, acc, s)
        plsc.subcore_barrier()

        def run(table):
            for ph in range(2):
                pltpu.sync_copy(src_hbm.at[2 * s + ph], src_v)
                pltpu.sync_copy(dst_hbm.at[2 * s + ph], dst_v)
                _edge_pipeline(table, acc, src_v, dst_v,
                               gbuf0, gbuf1, sg0, sg1, ss0, ss1, CH2)

        @pl.when(c == 0)
        def _():
            run(y_a)

        @pl.when(c == 1)
        def _():
            run(y_b)

        plsc.subcore_barrier()

        @pl.when(c == 0)
        def _():
            _writeout(acc, o_a, s)

        @pl.when(c == 1)
        def _():
            _writeout(acc, o_b, s)

    return seg


def _make_segsum_edgesplit():
    mesh = plsc.VectorSubcoreMesh(**_MESH)

    @functools.partial(
        pl.kernel,
        mesh=mesh,
        out_type=(
            jax.ShapeDtypeStruct((N, W), jnp.float32),
            jax.ShapeDtypeStruct((N, W), jnp.float32),
        ),
        scratch_types=[
            pltpu.VMEM_SHARED((NA, W), jnp.float32),
            pltpu.VMEM((CH2, K), jnp.int32),
            pltpu.VMEM((CH2, K), jnp.int32),
            pltpu.VMEM((K, W), jnp.float32),
            pltpu.VMEM((K, W), jnp.float32),
            pltpu.SemaphoreType.DMA,
            pltpu.SemaphoreType.DMA,
            pltpu.SemaphoreType.DMA,
            pltpu.SemaphoreType.DMA,
        ],
    )
    def seg(yp, src_hbm, dst_hbm, zer_hbm,
            o_a, o_b, acc, src_v, dst_v, gbuf0, gbuf1, sg0, sg1, ss0, ss1):
        c = lax.axis_index("c")
        s = lax.axis_index("s")
        w = c * NT + s

        pltpu.sync_copy(src_hbm.at[w], src_v)
        pltpu.sync_copy(dst_hbm.at[w], dst_v)
        _zero_acc(zer_hbm, acc, s)
        plsc.subcore_barrier()

        _edge_pipeline(yp, acc, src_v, dst_v, gbuf0, gbuf1,
                       sg0, sg1, ss0, ss1, CH2)

        plsc.subcore_barrier()

        @pl.when(c == 0)
        def _():
            _writeout(acc, o_a, s)

        @pl.when(c == 1)
        def _():
            _writeout(acc, o_b, s)

    return seg


def _make_count():
    mesh = plsc.VectorSubcoreMesh(**_MESH)

    @functools.partial(
        pl.kernel,
        mesh=mesh,
        out_type=(
            jax.ShapeDtypeStruct((N, W), jnp.float32),
            jax.ShapeDtypeStruct((N, W), jnp.float32),
        ),
        scratch_types=[
            pltpu.VMEM_SHARED((NA, W), jnp.float32),
            pltpu.VMEM((CH2, K), jnp.int32),
            pltpu.VMEM((K, W), jnp.float32),
        ],
    )
    def cnt(dst_hbm, ones_hbm, zer_hbm, o_a, o_b, acc, dst_v, ones_v):
        c = lax.axis_index("c")
        s = lax.axis_index("s")
        w = c * NT + s

        pltpu.sync_copy(dst_hbm.at[w], dst_v)
        pltpu.sync_copy(ones_hbm, ones_v)
        _zero_acc(zer_hbm, acc, s)
        plsc.subcore_barrier()

        def body(j, carry):
            pltpu.sync_copy(ones_v, acc.at[dst_v.at[j]], add=True)
            return carry
        lax.fori_loop(0, CH2, body, 0)

        plsc.subcore_barrier()

        @pl.when(c == 0)
        def _():
            _writeout(acc, o_a, s)

        @pl.when(c == 1)
        def _():
            _writeout(acc, o_b, s)

    return cnt


_seg_col = _make_segsum_colsplit()
_seg_edge = _make_segsum_edgesplit()
_count = _make_count()

_BR = 1000


def _tc_inv(ca, cb):
    def body(ca_ref, cb_ref, o_ref):
        cnt16 = ca_ref[:, 0:16] + cb_ref[:, 0:16]
        o_ref[...] = 1.0 / jnp.maximum(cnt16, 1.0)

    return pl.pallas_call(
        body,
        grid=(N // _BR,),
        in_specs=[
            pl.BlockSpec((_BR, W), lambda i: (i, 0)),
            pl.BlockSpec((_BR, W), lambda i: (i, 0)),
        ],
        out_specs=pl.BlockSpec((_BR, 16), lambda i: (i, 0)),
        out_shape=jax.ShapeDtypeStruct((N, 16), jnp.float32),
    )(ca, cb)


def _tc_in_y(x, Wl):
    D = Wl.shape[1]
    Wc = D // 2

    def body(x_ref, wl_ref, ya_ref, yb_ref):
        y = jnp.dot(x_ref[...], wl_ref[...],
                    preferred_element_type=jnp.float32)
        ya_ref[...] = y[:, :Wc]
        yb_ref[...] = y[:, Wc:]

    Din = x.shape[1]
    return pl.pallas_call(
        body,
        grid=(N // _BR,),
        in_specs=[
            pl.BlockSpec((_BR, Din), lambda i: (i, 0)),
            pl.BlockSpec((Din, D), lambda i: (0, 0)),
        ],
        out_specs=[
            pl.BlockSpec((_BR, Wc), lambda i: (i, 0)),
            pl.BlockSpec((_BR, Wc), lambda i: (i, 0)),
        ],
        out_shape=[
            jax.ShapeDtypeStruct((N, Wc), jnp.float32),
            jax.ShapeDtypeStruct((N, Wc), jnp.float32),
        ],
    )(x, Wl)


def _tc_r(h, Wr, b):
    D = Wr.shape[1]
    Din = h.shape[1]

    def body(h_ref, wr_ref, b_ref, r_ref):
        r_ref[...] = (jnp.dot(h_ref[...], wr_ref[...],
                              preferred_element_type=jnp.float32) + b_ref[...])

    return pl.pallas_call(
        body,
        grid=(N // _BR,),
        in_specs=[
            pl.BlockSpec((_BR, Din), lambda i: (i, 0)),
            pl.BlockSpec((Din, D), lambda i: (0, 0)),
            pl.BlockSpec((1, D), lambda i: (0, 0)),
        ],
        out_specs=pl.BlockSpec((_BR, D), lambda i: (i, 0)),
        out_shape=jax.ShapeDtypeStruct((N, D), jnp.float32),
    )(h, Wr, b.reshape(1, D))


def _tc_mid_y(aa, ab, inv, r_prev, Wl, pad_single):
    Dp = r_prev.shape[1]
    D = Wl.shape[1]
    Wc = D // 2

    def body(aa_ref, ab_ref, inv_ref, r_ref, wl_ref,
             ya_ref, yb_ref, h_ref):
        invv = inv_ref[:, 0:1]
        agg = jnp.concatenate([aa_ref[...], ab_ref[...]], axis=1)
        h = jnp.maximum(agg * invv + r_ref[...], 0.0)
        y = jnp.dot(h, wl_ref[...], preferred_element_type=jnp.float32)
        if pad_single:
            ya_ref[...] = jnp.concatenate(
                [y, jnp.zeros((_BR, W - D), jnp.float32)], axis=1)
            yb_ref[...] = jnp.zeros((_BR, 8), jnp.float32)
        else:
            ya_ref[...] = y[:, :Wc]
            yb_ref[...] = y[:, Wc:]
        h_ref[...] = h

    Wa = W if pad_single else Wc
    Wb = 8 if pad_single else Wc
    return pl.pallas_call(
        body,
        grid=(N // _BR,),
        in_specs=[
            pl.BlockSpec((_BR, Dp // 2), lambda i: (i, 0)),
            pl.BlockSpec((_BR, Dp // 2), lambda i: (i, 0)),
            pl.BlockSpec((_BR, 16), lambda i: (i, 0)),
            pl.BlockSpec((_BR, Dp), lambda i: (i, 0)),
            pl.BlockSpec((Dp, D), lambda i: (0, 0)),
        ],
        out_specs=[
            pl.BlockSpec((_BR, Wa), lambda i: (i, 0)),
            pl.BlockSpec((_BR, Wb), lambda i: (i, 0)),
            pl.BlockSpec((_BR, Dp), lambda i: (i, 0)),
        ],
        out_shape=[
            jax.ShapeDtypeStruct((N, Wa), jnp.float32),
            jax.ShapeDtypeStruct((N, Wb), jnp.float32),
            jax.ShapeDtypeStruct((N, Dp), jnp.float32),
        ],
    )(aa, ab, inv, r_prev, Wl)


def _tc_out(aa, ab, inv, r_prev):
    D = r_prev.shape[1]

    def body(aa_ref, ab_ref, inv_ref, r_ref, o_ref):
        agg = (aa_ref[...] + ab_ref[...])[:, :D]
        h = agg * inv_ref[:, 0:1] + r_ref[...]
        m = jnp.max(h, axis=1, keepdims=True)
        ex = jnp.exp(h - m)
        lse = jnp.log(jnp.sum(ex, axis=1, keepdims=True)) + m
        o_ref[...] = h - lse

    return pl.pallas_call(
        body,
        grid=(N // _BR,),
        in_specs=[
            pl.BlockSpec((_BR, W), lambda i: (i, 0)),
            pl.BlockSpec((_BR, W), lambda i: (i, 0)),
            pl.BlockSpec((_BR, 16), lambda i: (i, 0)),
            pl.BlockSpec((_BR, D), lambda i: (i, 0)),
        ],
        out_specs=pl.BlockSpec((_BR, D), lambda i: (i, 0)),
        out_shape=jax.ShapeDtypeStruct((N, D), jnp.float32),
    )(aa, ab, inv, r_prev)


def kernel(x, edge_index, W_l0, W_r0, b0, W_l1, W_r1, b1, W_l2, W_r2, b2):
    src = edge_index[0].astype(jnp.int32)
    dst = edge_index[1].astype(jnp.int32)
    pad = EPAD - E
    padv = jnp.arange(pad, dtype=jnp.int32)
    srcp = jnp.concatenate([src, padv % N])
    dstp = jnp.concatenate([dst, N + padv % (NA - N)])
    src_e = srcp.reshape(NT * NC, CH2, K)
    dst_e = dstp.reshape(NT * NC, CH2, K)

    zer = jnp.zeros((RPT, W), jnp.float32)
    ones = jnp.ones((K, W), jnp.float32)

    ca, cb = _count(dst_e, ones, zer)
    inv = _tc_inv(ca, cb)

    y0a, y0b = _tc_in_y(x, W_l0)
    a0a, a0b = _seg_col(y0a, y0b, src_e, dst_e, zer)
    r0 = _tc_r(x, W_r0, b0)
    y1a, y1b, h0 = _tc_mid_y(a0a, a0b, inv, r0, W_l1, pad_single=False)
    a1a, a1b = _seg_col(y1a, y1b, src_e, dst_e, zer)
    r1 = _tc_r(h0, W_r1, b1)
    y2p, _u2, h1 = _tc_mid_y(a1a, a1b, inv, r1, W_l2, pad_single=True)
    a2a, a2b = _seg_edge(y2p, src_e, dst_e, zer)
    r2 = _tc_r(h1, W_r2, b2)
    return _tc_out(a2a, a2b, inv, r2)


# speedup vs baseline: 1.0161x; 1.0161x over previous
import functools

import jax
import jax.numpy as jnp
from jax import lax
from jax.experimental import pallas as pl
from jax.experimental.pallas import tpu as pltpu
from jax.experimental.pallas import tpu_sc as plsc

N = 10000
E = 160000

NT = 16
NC = 2
K = 128
W = 128
EPAD = 163840
CH = EPAD // (NT * K)
CH2 = EPAD // (NT * NC * K)
NA = 10112
RPT = NA // NT
WPT = 624
TAIL = N - NT * WPT

_MESH = dict(core_axis_name="c", subcore_axis_name="s")


def _edge_pipeline(table, acc, src_v, dst_v, gbuf0, gbuf1,
                   sg0, sg1, ss0, ss1, nch):
    def fire_g(j, gbuf, sem):
        pltpu.async_copy(table.at[src_v.at[j]], gbuf, sem)

    def wait_g(j, gbuf, sem):
        pltpu.make_async_copy(table.at[src_v.at[j]], gbuf, sem).wait()

    def fire_s(j, gbuf, sem):
        pltpu.async_copy(gbuf, acc.at[dst_v.at[j]], sem, add=True)

    def wait_s(j, gbuf, sem):
        pltpu.make_async_copy(gbuf, acc.at[dst_v.at[j]], sem).wait()

    fire_g(0, gbuf0, sg0)

    def body(i, carry):
        j = 2 * i
        wait_g(j, gbuf0, sg0)
        fire_s(j, gbuf0, ss0)

        @pl.when(i > 0)
        def _():
            wait_s(j - 1, gbuf1, ss1)

        fire_g(j + 1, gbuf1, sg1)
        wait_g(j + 1, gbuf1, sg1)
        fire_s(j + 1, gbuf1, ss1)
        wait_s(j, gbuf0, ss0)

        @pl.when(i < nch // 2 - 1)
        def _():
            fire_g(j + 2, gbuf0, sg0)

        return carry

    lax.fori_loop(0, nch // 2, body, 0)
    wait_s(nch - 1, gbuf1, ss1)


def _zero_acc(zer_hbm, acc, s):
    pltpu.sync_copy(zer_hbm, acc.at[pl.ds(s * RPT, RPT)])


def _writeout(acc, o, s):
    w0 = s * WPT
    pltpu.sync_copy(acc.at[pl.ds(w0, WPT)], o.at[pl.ds(w0, WPT)])

    @pl.when(s == NT - 1)
    def _():
        pltpu.sync_copy(acc.at[pl.ds(NT * WPT, TAIL)],
                        o.at[pl.ds(NT * WPT, TAIL)])


def _make_segsum_colsplit():
    mesh = plsc.VectorSubcoreMesh(**_MESH)

    @functools.partial(
        pl.kernel,
        mesh=mesh,
        out_type=(
            jax.ShapeDtypeStruct((N, W), jnp.float32),
            jax.ShapeDtypeStruct((N, W), jnp.float32),
        ),
        scratch_types=[
            pltpu.VMEM_SHARED((NA, W), jnp.float32),
            pltpu.VMEM((CH2, K), jnp.int32),
            pltpu.VMEM((CH2, K), jnp.int32),
            pltpu.VMEM((K, W), jnp.float32),
            pltpu.VMEM((K, W), jnp.float32),
            pltpu.SemaphoreType.DMA,
            pltpu.SemaphoreType.DMA,
            pltpu.SemaphoreType.DMA,
            pltpu.SemaphoreType.DMA,
        ],
    )
    def seg(y_a, y_b, src_hbm, dst_hbm, zer_hbm,
            o_a, o_b, acc, src_v, dst_v, gbuf0, gbuf1, sg0, sg1, ss0, ss1):
        c = lax.axis_index("c")
        s = lax.axis_index("s")

        _zero_acc(zer_hbm, acc, s)
        plsc.subcore_barrier()

        def run(table):
            for ph in range(2):
                pltpu.sync_copy(src_hbm.at[2 * s + ph], src_v)
                pltpu.sync_copy(dst_hbm.at[2 * s + ph], dst_v)
                _edge_pipeline(table, acc, src_v, dst_v,
                               gbuf0, gbuf1, sg0, sg1, ss0, ss1, CH2)

        @pl.when(c == 0)
        def _():
            run(y_a)

        @pl.when(c == 1)
        def _():
            run(y_b)

        plsc.subcore_barrier()

        @pl.when(c == 0)
        def _():
            _writeout(acc, o_a, s)

        @pl.when(c == 1)
        def _():
            _writeout(acc, o_b, s)

    return seg


def _make_segsum_edgesplit():
    mesh = plsc.VectorSubcoreMesh(**_MESH)

    @functools.partial(
        pl.kernel,
        mesh=mesh,
        out_type=(
            jax.ShapeDtypeStruct((N, W), jnp.float32),
            jax.ShapeDtypeStruct((N, W), jnp.float32),
        ),
        scratch_types=[
            pltpu.VMEM_SHARED((NA, W), jnp.float32),
            pltpu.VMEM((CH2, K), jnp.int32),
            pltpu.VMEM((CH2, K), jnp.int32),
            pltpu.VMEM((K, W), jnp.float32),
            pltpu.VMEM((K, W), jnp.float32),
            pltpu.SemaphoreType.DMA,
            pltpu.SemaphoreType.DMA,
            pltpu.SemaphoreType.DMA,
            pltpu.SemaphoreType.DMA,
        ],
    )
    def seg(yp, src_hbm, dst_hbm, zer_hbm,
            o_a, o_b, acc, src_v, dst_v, gbuf0, gbuf1, sg0, sg1, ss0, ss1):
        c = lax.axis_index("c")
        s = lax.axis_index("s")
        w = c * NT + s

        pltpu.sync_copy(src_hbm.at[w], src_v)
        pltpu.sync_copy(dst_hbm.at[w], dst_v)
        _zero_acc(zer_hbm, acc, s)
        plsc.subcore_barrier()

        _edge_pipeline(yp, acc, src_v, dst_v, gbuf0, gbuf1,
                       sg0, sg1, ss0, ss1, CH2)

        plsc.subcore_barrier()

        @pl.when(c == 0)
        def _():
            _writeout(acc, o_a, s)

        @pl.when(c == 1)
        def _():
            _writeout(acc, o_b, s)

    return seg


def _make_count():
    mesh = plsc.VectorSubcoreMesh(**_MESH)

    @functools.partial(
        pl.kernel,
        mesh=mesh,
        out_type=(
            jax.ShapeDtypeStruct((N, W), jnp.float32),
            jax.ShapeDtypeStruct((N, W), jnp.float32),
        ),
        scratch_types=[
            pltpu.VMEM_SHARED((NA, W), jnp.float32),
            pltpu.VMEM((CH2, K), jnp.int32),
            pltpu.VMEM((K, W), jnp.float32),
        ],
    )
    def cnt(dst_hbm, ones_hbm, zer_hbm, o_a, o_b, acc, dst_v, ones_v):
        c = lax.axis_index("c")
        s = lax.axis_index("s")
        w = c * NT + s

        pltpu.sync_copy(dst_hbm.at[w], dst_v)
        pltpu.sync_copy(ones_hbm, ones_v)
        _zero_acc(zer_hbm, acc, s)
        plsc.subcore_barrier()

        def body(j, carry):
            pltpu.sync_copy(ones_v, acc.at[dst_v.at[j]], add=True)
            return carry
        lax.fori_loop(0, CH2, body, 0)

        plsc.subcore_barrier()

        @pl.when(c == 0)
        def _():
            _writeout(acc, o_a, s)

        @pl.when(c == 1)
        def _():
            _writeout(acc, o_b, s)

    return cnt


_seg_col = _make_segsum_colsplit()
_seg_edge = _make_segsum_edgesplit()
_count = _make_count()

_BR = 1000


def _tc_inv(ca, cb):
    def body(ca_ref, cb_ref, o_ref):
        cnt16 = ca_ref[:, 0:16] + cb_ref[:, 0:16]
        o_ref[...] = 1.0 / jnp.maximum(cnt16, 1.0)

    return pl.pallas_call(
        body,
        grid=(N // _BR,),
        in_specs=[
            pl.BlockSpec((_BR, W), lambda i: (i, 0)),
            pl.BlockSpec((_BR, W), lambda i: (i, 0)),
        ],
        out_specs=pl.BlockSpec((_BR, 16), lambda i: (i, 0)),
        out_shape=jax.ShapeDtypeStruct((N, 16), jnp.float32),
    )(ca, cb)


def _tc_in(x, Wl, Wr, b):
    D = Wl.shape[1]
    Wc = D // 2

    def body(x_ref, wl_ref, wr_ref, b_ref, ya_ref, yb_ref, r_ref):
        xb = x_ref[...]
        y = jnp.dot(xb, wl_ref[...], preferred_element_type=jnp.float32)
        ya_ref[...] = y[:, :Wc]
        yb_ref[...] = y[:, Wc:]
        r_ref[...] = (jnp.dot(xb, wr_ref[...],
                              preferred_element_type=jnp.float32) + b_ref[...])

    Din = x.shape[1]
    return pl.pallas_call(
        body,
        grid=(N // _BR,),
        in_specs=[
            pl.BlockSpec((_BR, Din), lambda i: (i, 0)),
            pl.BlockSpec((Din, D), lambda i: (0, 0)),
            pl.BlockSpec((Din, D), lambda i: (0, 0)),
            pl.BlockSpec((1, D), lambda i: (0, 0)),
        ],
        out_specs=[
            pl.BlockSpec((_BR, Wc), lambda i: (i, 0)),
            pl.BlockSpec((_BR, Wc), lambda i: (i, 0)),
            pl.BlockSpec((_BR, D), lambda i: (i, 0)),
        ],
        out_shape=[
            jax.ShapeDtypeStruct((N, Wc), jnp.float32),
            jax.ShapeDtypeStruct((N, Wc), jnp.float32),
            jax.ShapeDtypeStruct((N, D), jnp.float32),
        ],
    )(x, Wl, Wr, b.reshape(1, D))


def _tc_mid(aa, ab, inv, r_prev, Wl, Wr, b, pad_single):
    Dp = r_prev.shape[1]
    D = Wl.shape[1]
    Wc = D // 2

    def body(aa_ref, ab_ref, inv_ref, r_ref, wl_ref, wr_ref, b_ref,
             ya_ref, yb_ref, ro_ref):
        invv = inv_ref[:, 0:1]
        agg = jnp.concatenate([aa_ref[...], ab_ref[...]], axis=1)
        h = jnp.maximum(agg * invv + r_ref[...], 0.0)
        y = jnp.dot(h, wl_ref[...], preferred_element_type=jnp.float32)
        if pad_single:
            ya_ref[...] = jnp.concatenate(
                [y, jnp.zeros((_BR, W - D), jnp.float32)], axis=1)
            yb_ref[...] = jnp.zeros((_BR, 8), jnp.float32)
        else:
            ya_ref[...] = y[:, :Wc]
            yb_ref[...] = y[:, Wc:]
        ro_ref[...] = (jnp.dot(h, wr_ref[...],
                               preferred_element_type=jnp.float32) + b_ref[...])

    Wa = W if pad_single else Wc
    Wb = 8 if pad_single else Wc
    return pl.pallas_call(
        body,
        grid=(N // _BR,),
        in_specs=[
            pl.BlockSpec((_BR, Dp // 2), lambda i: (i, 0)),
            pl.BlockSpec((_BR, Dp // 2), lambda i: (i, 0)),
            pl.BlockSpec((_BR, 16), lambda i: (i, 0)),
            pl.BlockSpec((_BR, Dp), lambda i: (i, 0)),
            pl.BlockSpec((Dp, D), lambda i: (0, 0)),
            pl.BlockSpec((Dp, D), lambda i: (0, 0)),
            pl.BlockSpec((1, D), lambda i: (0, 0)),
        ],
        out_specs=[
            pl.BlockSpec((_BR, Wa), lambda i: (i, 0)),
            pl.BlockSpec((_BR, Wb), lambda i: (i, 0)),
            pl.BlockSpec((_BR, D), lambda i: (i, 0)),
        ],
        out_shape=[
            jax.ShapeDtypeStruct((N, Wa), jnp.float32),
            jax.ShapeDtypeStruct((N, Wb), jnp.float32),
            jax.ShapeDtypeStruct((N, D), jnp.float32),
        ],
    )(aa, ab, inv, r_prev, Wl, Wr, b.reshape(1, D))


def _tc_out(aa, ab, inv, r_prev):
    D = r_prev.shape[1]

    def body(aa_ref, ab_ref, inv_ref, r_ref, o_ref):
        agg = (aa_ref[...] + ab_ref[...])[:, :D]
        h = agg * inv_ref[:, 0:1] + r_ref[...]
        m = jnp.max(h, axis=1, keepdims=True)
        ex = jnp.exp(h - m)
        lse = jnp.log(jnp.sum(ex, axis=1, keepdims=True)) + m
        o_ref[...] = h - lse

    return pl.pallas_call(
        body,
        grid=(N // _BR,),
        in_specs=[
            pl.BlockSpec((_BR, W), lambda i: (i, 0)),
            pl.BlockSpec((_BR, W), lambda i: (i, 0)),
            pl.BlockSpec((_BR, 16), lambda i: (i, 0)),
            pl.BlockSpec((_BR, D), lambda i: (i, 0)),
        ],
        out_specs=pl.BlockSpec((_BR, D), lambda i: (i, 0)),
        out_shape=jax.ShapeDtypeStruct((N, D), jnp.float32),
    )(aa, ab, inv, r_prev)


def kernel(x, edge_index, W_l0, W_r0, b0, W_l1, W_r1, b1, W_l2, W_r2, b2):
    src = edge_index[0].astype(jnp.int32)
    dst = edge_index[1].astype(jnp.int32)
    pad = EPAD - E
    padv = jnp.arange(pad, dtype=jnp.int32)
    srcp = jnp.concatenate([src, padv % N])
    dstp = jnp.concatenate([dst, N + padv % (NA - N)])
    src_e = srcp.reshape(NT * NC, CH2, K)
    dst_e = dstp.reshape(NT * NC, CH2, K)

    zer = jnp.zeros((RPT, W), jnp.float32)
    ones = jnp.ones((K, W), jnp.float32)

    ca, cb = _count(dst_e, ones, zer)
    inv = _tc_inv(ca, cb)

    y0a, y0b, r0 = _tc_in(x, W_l0, W_r0, b0)
    a0a, a0b = _seg_col(y0a, y0b, src_e, dst_e, zer)
    y1a, y1b, r1 = _tc_mid(a0a, a0b, inv, r0, W_l1, W_r1, b1,
                           pad_single=False)
    a1a, a1b = _seg_col(y1a, y1b, src_e, dst_e, zer)
    y2p, _u2, r2 = _tc_mid(a1a, a1b, inv, r1, W_l2, W_r2, b2,
                           pad_single=True)
    a2a, a2b = _seg_edge(y2p, src_e, dst_e, zer)
    return _tc_out(a2a, a2b, inv, r2)
